# baseline (speedup 1.0000x reference)
import functools

import jax
import jax.numpy as jnp
import numpy as np
from jax import lax
from jax.experimental import pallas as pl
from jax.experimental.pallas import tpu as pltpu
from jax.experimental.pallas import tpu_sc as plsc

_L = 16
_CB = 128
_NSLOT = 8
_PAD = 3


@jax.jit
def kernel(x_num, x_cat, offsets, table):
    B, NN = x_num.shape
    F = x_cat.shape[1]
    V, D = table.shape
    POUTW = _PAD + NN + F * D
    NHEAD = _PAD + NN

    info = plsc.get_sparse_core_info()
    NC, NS = info.num_cores, info.num_subcores
    NW = NC * NS
    assert B % (NW * _CB) == 0
    rows_per_w = B // NW
    n_chunks = rows_per_w // _CB

    p = np.arange(_CB * NN)
    t_srow = jnp.asarray(p // NN, jnp.int32)
    t_scol = jnp.asarray(_PAD + p % NN, jnp.int32)
    off_bcast = jnp.tile(offsets[:, None], (1, _L))

    mesh = plsc.VectorSubcoreMesh(core_axis_name="c", subcore_axis_name="s")

    scratch = [
        pltpu.VMEM((F, _CB), jnp.int32),
        pltpu.VMEM((F, _CB), jnp.int32),
        pltpu.VMEM((F, _L), jnp.int32),
        pltpu.VMEM((_NSLOT, _CB, D), jnp.float32),
        pltpu.VMEM((_CB * NN,), jnp.float32),
        pltpu.VMEM((_CB, NHEAD), jnp.float32),
        pltpu.VMEM((_CB * NN,), jnp.int32),
        pltpu.VMEM((_CB * NN,), jnp.int32),
    ] + [pltpu.SemaphoreType.DMA] * (2 * _NSLOT + 1)

    @functools.partial(
        pl.kernel,
        out_type=jax.ShapeDtypeStruct((B, POUTW), jnp.float32),
        mesh=mesh,
        scratch_types=scratch,
        compiler_params=pltpu.CompilerParams(
            use_tc_tiling_on_sc=False, needs_layout_passes=False),
    )
    def run(xnum_hbm, xcatT_hbm, off_hbm, table_hbm, srow_hbm, scol_hbm,
            out_hbm,
            xcatT_v, idxT_v, off_v, slots, xnum_v, head_v,
            srow_v, scol_v, *sems):
        sg = sems[:_NSLOT]
        sw = sems[_NSLOT:2 * _NSLOT]
        sem_h = sems[2 * _NSLOT]
        cid = lax.axis_index("c")
        sid = lax.axis_index("s")
        wid = sid * NC + cid

        pltpu.sync_copy(off_hbm, off_v)
        pltpu.sync_copy(srow_hbm, srow_v)
        pltpu.sync_copy(scol_hbm, scol_v)

        def gather_cp(f):
            return pltpu.make_async_copy(
                table_hbm.at[idxT_v.at[f]], slots.at[f % _NSLOT],
                sg[f % _NSLOT])

        def write_cp(f, base):
            return pltpu.make_async_copy(
                slots.at[f % _NSLOT],
                out_hbm.at[pl.ds(base, _CB), pl.ds(NHEAD + D * f, D)],
                sw[f % _NSLOT])

        def head_cp(base):
            return pltpu.make_async_copy(
                head_v, out_hbm.at[pl.ds(base, _CB), pl.ds(0, NHEAD)], sem_h)

        def chunk_body(c, carry):
            base = wid * rows_per_w + c * _CB

            pltpu.sync_copy(xcatT_hbm.at[:, pl.ds(base, _CB)], xcatT_v)
            pltpu.sync_copy(xnum_hbm.at[wid * n_chunks + c], xnum_v)
            for f in range(F):
                off_vec = off_v[f, :]
                for j in range(_CB // _L):
                    s = pl.ds(j * _L, _L)
                    idxT_v[f, s] = xcatT_v[f, s] + off_vec

            for k in range(_CB * NN // _L):
                s = pl.ds(k * _L, _L)
                plsc.store_scatter(head_v, [srow_v[s], scol_v[s]], xnum_v[s])
            head_cp(base).start()

            for f in range(_NSLOT):
                gather_cp(f).start()
            for f in range(F):
                gather_cp(f).wait()
                write_cp(f, base).start()
                nf = f + _NSLOT
                if nf < F:
                    write_cp(f, base).wait()
                    gather_cp(nf).start()
            for f in range(F - _NSLOT, F):
                write_cp(f, base).wait()
            head_cp(base).wait()
            return carry

        lax.fori_loop(0, n_chunks, chunk_body, 0)

    out_padded = run(x_num.reshape(B // _CB, _CB * NN), x_cat.T, off_bcast,
                     table, t_srow, t_scol)
    return out_padded[:, _PAD:]
